# baseline (speedup 1.0000x reference)
import jax
import jax.numpy as jnp
from jax import lax
from jax.experimental import pallas as pl
from jax.experimental.pallas import tpu as pltpu
from jax.experimental.pallas import tpu_sc as plsc

N = 10000
E = 320000
D = 128
NC = 2
NS = 16
NW = NC * NS
CHUNK = 80
NCHUNK = 125
NPAIR = (NCHUNK - 1) // 2
N2 = 10240
RPT = N2 // NS

_mesh = plsc.VectorSubcoreMesh(core_axis_name="c", subcore_axis_name="s")


def _fill(ref, row):
    def fr(i, _):
        for j in range(D // 16):
            ref[i, pl.ds(j * 16, 16)] = row
        return 0
    lax.fori_loop(0, CHUNK, fr, 0)


def _sc_agg(with_deg):
    out_type = [jax.ShapeDtypeStruct((NC, N2, D), jnp.float32)]
    if with_deg:
        out_type.append(jax.ShapeDtypeStruct((NC, N2, D), jnp.float32))
    scratch = [
        pltpu.VMEM((NCHUNK * CHUNK,), jnp.int32),
        pltpu.VMEM((NCHUNK, CHUNK), jnp.int32),
        pltpu.VMEM((CHUNK, D), jnp.float32),
        pltpu.VMEM((CHUNK, D), jnp.float32),
        pltpu.VMEM_SHARED((N2, D), jnp.float32),
        pltpu.SemaphoreType.DMA,
        pltpu.SemaphoreType.DMA,
        pltpu.SemaphoreType.DMA,
    ]

    def body(xs, esrc, edst, *rest):
        if with_deg:
            acc_out, deg_out, src_v, dst_v, buf0, buf1, acc_sh, si, sg0, sg1 = rest
        else:
            acc_out, src_v, dst_v, buf0, buf1, acc_sh, si, sg0, sg1 = rest
        cid = lax.axis_index("c")
        sid = lax.axis_index("s")
        wid = cid * NS + sid
        bufs = (buf0, buf1)
        sgs = (sg0, sg1)

        def gather(c, p):
            pltpu.async_copy(xs.at[src_v.at[pl.ds(c * CHUNK, CHUNK)]],
                             bufs[p], sgs[p])

        def wait_gather(p):
            pltpu.make_async_copy(xs.at[src_v.at[pl.ds(0, CHUNK)]],
                                  bufs[p], sgs[p]).wait()

        def zero_acc():
            _fill(buf0, jnp.zeros((16,), jnp.float32))
            for k in range(RPT // CHUNK):
                pltpu.sync_copy(buf0, acc_sh.at[pl.ds(sid * RPT + k * CHUNK, CHUNK)])
            plsc.subcore_barrier()

        def export(out):
            plsc.subcore_barrier()
            pltpu.sync_copy(acc_sh.at[pl.ds(sid * RPT, RPT)],
                            out.at[cid, pl.ds(sid * RPT, RPT)])

        pltpu.async_copy(esrc.at[wid], src_v, si)
        pltpu.async_copy(edst.at[wid], dst_v, si)
        zero_acc()
        pltpu.make_async_copy(esrc.at[wid], src_v, si).wait()
        pltpu.make_async_copy(edst.at[wid], dst_v, si).wait()

        gather(0, 0)
        gather(1, 1)

        def pair(jj, _):
            for t in range(2):
                c = 2 * jj + t
                wait_gather(t)
                pltpu.sync_copy(bufs[t], acc_sh.at[dst_v.at[c]], add=True)
                if t == 0:
                    gather(c + 2, t)
                else:
                    @pl.when(jj < NPAIR - 1)
                    def _():
                        gather(c + 2, t)
            return 0
        lax.fori_loop(0, NPAIR, pair, 0)
        wait_gather(0)
        pltpu.sync_copy(buf0, acc_sh.at[dst_v.at[NCHUNK - 1]], add=True)
        export(acc_out)

        if with_deg:
            zero_acc()
            _fill(buf0, jnp.ones((16,), jnp.float32))

            def dstep(c, _):
                pltpu.sync_copy(buf0, acc_sh.at[dst_v.at[c]], add=True)
                return 0
            lax.fori_loop(0, NCHUNK, dstep, 0)
            export(deg_out)

    return pl.kernel(body, out_type=out_type, mesh=_mesh,
                     scratch_types=scratch)


_sc_agg_deg = _sc_agg(True)
_sc_agg_only = _sc_agg(False)

_TC_R = 1000


def _tc_body(a_ref, dg_ref, x_ref, wl_ref, wr_ref, b_ref, o_ref):
    agg = a_ref[0] + a_ref[1]
    deg = dg_ref[0] + dg_ref[1]
    mean = agg / jnp.maximum(deg, 1.0)
    o_ref[...] = jnp.maximum(
        jnp.dot(mean, wl_ref[...], preferred_element_type=jnp.float32)
        + jnp.dot(x_ref[...], wr_ref[...], preferred_element_type=jnp.float32)
        + b_ref[...], 0.0)


def _tc_layer(a, dg, xs, wl, wr, b2d):
    return pl.pallas_call(
        _tc_body,
        grid=(N // _TC_R,),
        in_specs=[
            pl.BlockSpec((NC, _TC_R, D), lambda i: (0, i, 0)),
            pl.BlockSpec((NC, _TC_R, D), lambda i: (0, i, 0)),
            pl.BlockSpec((_TC_R, D), lambda i: (i, 0)),
            pl.BlockSpec((D, D), lambda i: (0, 0)),
            pl.BlockSpec((D, D), lambda i: (0, 0)),
            pl.BlockSpec((1, D), lambda i: (0, 0)),
        ],
        out_specs=pl.BlockSpec((_TC_R, D), lambda i: (i, 0)),
        out_shape=jax.ShapeDtypeStruct((N, D), jnp.float32),
    )(a, dg, xs, wl, wr, b2d)


def kernel(x, edge_index, W_l1, W_r1, b1, W_l2, W_r2, b2):
    ei = edge_index.astype(jnp.int32)
    esrc = ei[0].reshape(NW, NCHUNK * CHUNK)
    edst = ei[1].reshape(NW, NCHUNK, CHUNK)

    a1, deg = _sc_agg_deg(x, esrc, edst)
    h = _tc_layer(a1, deg, x, W_l1, W_r1, b1.reshape(1, D))
    (a2,) = _sc_agg_only(h, esrc, edst)
    out = _tc_layer(a2, deg, h, W_l2, W_r2, b2.reshape(1, D))
    return out


# speedup vs baseline: 1.0292x; 1.0292x over previous
import jax
import jax.numpy as jnp
from jax import lax
from jax.experimental import pallas as pl
from jax.experimental.pallas import tpu as pltpu
from jax.experimental.pallas import tpu_sc as plsc

N = 10000
E = 320000
D = 128
NC = 2
NS = 16
NW = NC * NS
CHUNK = 128
GRP = 8
NGRP = 10
NPAIR = NGRP // 2
NCHUNK = NGRP * GRP
EP = NW * NCHUNK * CHUNK
N2 = 10240
RPT = N2 // NS

_mesh = plsc.VectorSubcoreMesh(core_axis_name="c", subcore_axis_name="s")


def _fill(ref, row):
    def fr(i, _):
        for j in range(D // 16):
            ref[i, pl.ds(j * 16, 16)] = row
        return 0
    lax.fori_loop(0, CHUNK, fr, 0)


def _sc_agg(with_deg):
    out_type = [jax.ShapeDtypeStruct((NC, N2, D), jnp.float32)]
    if with_deg:
        out_type.append(jax.ShapeDtypeStruct((NC, N2, D), jnp.float32))
    scratch = [
        pltpu.VMEM((GRP, 2, CHUNK), jnp.int32),
        pltpu.VMEM((GRP, 2, CHUNK), jnp.int32),
        pltpu.VMEM((CHUNK, D), jnp.float32),
        pltpu.VMEM((CHUNK, D), jnp.float32),
        pltpu.VMEM_SHARED((N2, D), jnp.float32),
        pltpu.SemaphoreType.DMA,
        pltpu.SemaphoreType.DMA,
        pltpu.SemaphoreType.DMA,
        pltpu.SemaphoreType.DMA,
    ]

    def body(xp, eidx, *rest):
        if with_deg:
            acc_out, deg_out, idxA, idxB, buf0, buf1, acc_sh, siA, siB, sg0, sg1 = rest
        else:
            acc_out, idxA, idxB, buf0, buf1, acc_sh, siA, siB, sg0, sg1 = rest
        cid = lax.axis_index("c")
        sid = lax.axis_index("s")
        wid = cid * NS + sid
        bufs = (buf0, buf1)
        sgs = (sg0, sg1)
        idxs = (idxA, idxB)
        sis = (siA, siB)

        def load_idx_async(g, which):
            pltpu.async_copy(eidx.at[wid, g], idxs[which], sis[which])

        def wait_idx(which):
            pltpu.make_async_copy(eidx.at[wid, 0], idxs[which], sis[which]).wait()

        def gather(ia, mi, p):
            pltpu.async_copy(xp.at[ia.at[mi, 0]], bufs[p], sgs[p])

        def wait_gather(ia, mi, p):
            pltpu.make_async_copy(xp.at[ia.at[mi, 0]], bufs[p], sgs[p]).wait()

        def zero_acc():
            _fill(buf0, jnp.zeros((16,), jnp.float32))
            for k in range(RPT // CHUNK):
                pltpu.sync_copy(buf0, acc_sh.at[pl.ds(sid * RPT + k * CHUNK, CHUNK)])
            plsc.subcore_barrier()

        def export(out):
            plsc.subcore_barrier()
            pltpu.sync_copy(acc_sh.at[pl.ds(sid * RPT, RPT)],
                            out.at[cid, pl.ds(sid * RPT, RPT)])

        zero_acc()
        pltpu.sync_copy(eidx.at[wid, 0], idxA)
        load_idx_async(1, 1)
        gather(idxA, 0, 0)
        gather(idxA, 1, 1)

        def pair(gg, _):
            g0 = 2 * gg
            for m in range(2 * GRP):
                p = m & 1
                ia, mi = (idxA, m) if m < GRP else (idxB, m - GRP)
                wait_gather(ia, mi, p)
                pltpu.sync_copy(bufs[p], acc_sh.at[ia.at[mi, 1]], add=True)
                if m == GRP - 2:
                    wait_idx(1)
                if m == GRP - 1:
                    @pl.when(gg < NPAIR - 1)
                    def _():
                        load_idx_async(g0 + 2, 0)
                m2 = m + 2
                if m2 < GRP:
                    gather(idxA, m2, p)
                elif m2 < 2 * GRP:
                    gather(idxB, m2 - GRP, p)
                elif m2 == 2 * GRP:
                    @pl.when(gg < NPAIR - 1)
                    def _():
                        wait_idx(0)
                        gather(idxA, 0, p)
                else:
                    @pl.when(gg < NPAIR - 1)
                    def _():
                        gather(idxA, 1, p)
                        load_idx_async(g0 + 3, 1)
            return 0
        lax.fori_loop(0, NPAIR, pair, 0)
        export(acc_out)

        if with_deg:
            zero_acc()
            _fill(buf0, jnp.ones((16,), jnp.float32))
            pltpu.sync_copy(eidx.at[wid, 0], idxA)
            load_idx_async(1, 1)

            def dpair(gg, _):
                g0 = 2 * gg
                for m in range(2 * GRP):
                    ia, mi = (idxA, m) if m < GRP else (idxB, m - GRP)
                    pltpu.sync_copy(buf0, acc_sh.at[ia.at[mi, 1]], add=True)
                    if m == GRP - 1:
                        wait_idx(1)
                        @pl.when(gg < NPAIR - 1)
                        def _():
                            load_idx_async(g0 + 2, 0)
                    if m == 2 * GRP - 1:
                        @pl.when(gg < NPAIR - 1)
                        def _():
                            load_idx_async(g0 + 3, 1)
                            wait_idx(0)
                return 0
            lax.fori_loop(0, NPAIR, dpair, 0)
            export(deg_out)

    return pl.kernel(body, out_type=out_type, mesh=_mesh,
                     scratch_types=scratch)


_sc_agg_deg = _sc_agg(True)
_sc_agg_only = _sc_agg(False)

_TC_R = 1280


def _tc_body(a_ref, dg_ref, x_ref, wl_ref, wr_ref, b_ref, o_ref):
    agg = a_ref[0] + a_ref[1]
    deg = dg_ref[0] + dg_ref[1]
    mean = agg / jnp.maximum(deg, 1.0)
    o_ref[...] = jnp.maximum(
        jnp.dot(mean, wl_ref[...], preferred_element_type=jnp.float32)
        + jnp.dot(x_ref[...], wr_ref[...], preferred_element_type=jnp.float32)
        + b_ref[...], 0.0)


def _tc_layer(a, dg, xp, wl, wr, b2d):
    return pl.pallas_call(
        _tc_body,
        grid=(N2 // _TC_R,),
        in_specs=[
            pl.BlockSpec((NC, _TC_R, D), lambda i: (0, i, 0)),
            pl.BlockSpec((NC, _TC_R, D), lambda i: (0, i, 0)),
            pl.BlockSpec((_TC_R, D), lambda i: (i, 0)),
            pl.BlockSpec((D, D), lambda i: (0, 0)),
            pl.BlockSpec((D, D), lambda i: (0, 0)),
            pl.BlockSpec((1, D), lambda i: (0, 0)),
        ],
        out_specs=pl.BlockSpec((_TC_R, D), lambda i: (i, 0)),
        out_shape=jax.ShapeDtypeStruct((N2, D), jnp.float32),
    )(a, dg, xp, wl, wr, b2d)


def kernel(x, edge_index, W_l1, W_r1, b1, W_l2, W_r2, b2):
    src = edge_index[0].astype(jnp.int32)
    dst = edge_index[1].astype(jnp.int32)
    npad = EP - E
    pad_src = (jnp.arange(npad, dtype=jnp.int32) * 97) % N
    pad_dst = N + (jnp.arange(npad, dtype=jnp.int32) % NW)
    srcp = jnp.concatenate([src, pad_src]).reshape(NW, NGRP, GRP, CHUNK)
    dstp = jnp.concatenate([dst, pad_dst]).reshape(NW, NGRP, GRP, CHUNK)
    eidx = jnp.stack([srcp, dstp], axis=3)
    xp = jnp.zeros((N2, D), jnp.float32).at[:N].set(x)

    a1, deg = _sc_agg_deg(xp, eidx)
    h = _tc_layer(a1, deg, xp, W_l1, W_r1, b1.reshape(1, D))
    (a2,) = _sc_agg_only(h, eidx)
    out = _tc_layer(a2, deg, h, W_l2, W_r2, b2.reshape(1, D))
    return out[:N]
